# Initial kernel scaffold; baseline (speedup 1.0000x reference)
import jax
import jax.numpy as jnp
from jax.experimental import pallas as pl


def kernel(char_indices, word_embedding, emb_tbl, w_ih, w_hh, bias, w_lw, w_lh, b_lin):
    raise NotImplementedError("write your pallas kernel here")


# speedup vs baseline: 4.4065x; 4.4065x over previous
import jax
import jax.numpy as jnp
from jax import lax
from jax.experimental import pallas as pl
from jax.experimental.pallas import tpu as pltpu

_BLOCK = 512
_NCH = 4


def _encoder_kernel(idx_ref, wemb_ref, wtop_ref, bias_ref, wout_ref,
                    blin_ref, out_ref):
    BLK, T = idx_ref.shape
    VH, H4 = wtop_ref.shape
    H = H4 // 4
    V = VH - H
    Dw = out_ref.shape[1]
    SB = BLK // _NCH

    idx = idx_ref[...]
    wtop = wtop_ref[...]
    bias = bias_ref[...]
    lane_iota = lax.broadcasted_iota(jnp.int32, (SB, V), 1)

    col = lax.broadcasted_iota(jnp.int32, (1, H4), 1)
    is_g = (col >= 2 * H) & (col < 3 * H)
    qs = jnp.where(is_g, 1.0, 0.5).astype(jnp.float32)
    qb = jnp.where(is_g, 0.0, 0.5).astype(jnp.float32)

    hs = [jnp.zeros((SB, H), jnp.float32) for _ in range(_NCH)]
    cs = [jnp.zeros((SB, H), jnp.float32) for _ in range(_NCH)]

    for t in range(T):
        for ch in range(_NCH):
            rows = slice(ch * SB, (ch + 1) * SB)
            onehot = (idx[rows, t:t + 1] == lane_iota).astype(jnp.float32)
            lhs = jnp.concatenate([onehot, hs[ch]], axis=1)
            pre = jnp.dot(lhs, wtop,
                          preferred_element_type=jnp.float32) + bias
            act = jnp.tanh(pre) * qs + qb
            i_g = act[:, 0 * H:1 * H]
            f_g = act[:, 1 * H:2 * H]
            g_g = act[:, 2 * H:3 * H]
            o_g = act[:, 3 * H:4 * H]
            cs[ch] = f_g * cs[ch] + i_g * g_g
            hs[ch] = o_g * jnp.tanh(cs[ch])

    wout = wout_ref[...]
    blin = blin_ref[...]
    for ch in range(_NCH):
        rows = slice(ch * SB, (ch + 1) * SB)
        comb = jnp.concatenate([wemb_ref[rows, :], hs[ch]], axis=1)
        res = jnp.dot(comb, wout, preferred_element_type=jnp.float32) + blin
        out_ref[rows, :] = jnp.maximum(res, 0.0)


def kernel(char_indices, word_embedding, emb_tbl, w_ih, w_hh, bias,
           w_lw, w_lh, b_lin):
    B, T = char_indices.shape
    Dw = word_embedding.shape[1]
    H4 = w_ih.shape[0]
    H = H4 // 4
    V = emb_tbl.shape[0]

    colw = jnp.arange(H4)
    sc = jnp.where((colw >= 2 * H) & (colw < 3 * H), 1.0, 0.5)
    sc = sc.astype(jnp.float32)[None, :]

    folded = emb_tbl @ w_ih.T
    wtop = jnp.concatenate([folded, w_hh.T], axis=0) * sc
    bias_sc = bias * sc
    wout = jnp.concatenate([w_lw, w_lh], axis=1).T

    n_blk = (B + _BLOCK - 1) // _BLOCK
    Bp = n_blk * _BLOCK
    if Bp != B:
        char_indices = jnp.pad(char_indices, ((0, Bp - B), (0, 0)))
        word_embedding = jnp.pad(word_embedding, ((0, Bp - B), (0, 0)))

    out = pl.pallas_call(
        _encoder_kernel,
        out_shape=jax.ShapeDtypeStruct((Bp, Dw), jnp.float32),
        grid=(n_blk,),
        in_specs=[
            pl.BlockSpec((_BLOCK, T), lambda i: (i, 0)),
            pl.BlockSpec((_BLOCK, Dw), lambda i: (i, 0)),
            pl.BlockSpec((V + H, H4), lambda i: (0, 0)),
            pl.BlockSpec((1, H4), lambda i: (0, 0)),
            pl.BlockSpec((Dw + H, Dw), lambda i: (0, 0)),
            pl.BlockSpec((1, Dw), lambda i: (0, 0)),
        ],
        out_specs=pl.BlockSpec((_BLOCK, Dw), lambda i: (i, 0)),
        compiler_params=pltpu.CompilerParams(
            dimension_semantics=("parallel",)),
    )(char_indices, word_embedding, wtop, bias_sc, wout, b_lin)
    return out[:B]


# speedup vs baseline: 4.4367x; 1.0069x over previous
import jax
import jax.numpy as jnp
from jax import lax
from jax.experimental import pallas as pl
from jax.experimental.pallas import tpu as pltpu

_BLOCK = 512
_NCH = 4


def _encoder_kernel(idx_ref, wemb_ref, wtop_ref, wout_ref,
                    blin_ref, out_ref):
    BLK, T = idx_ref.shape
    VH, H4 = wtop_ref.shape
    H = H4 // 4
    V = VH - H
    Dw = out_ref.shape[1]
    SB = BLK // _NCH

    idx = idx_ref[...]
    wtop = wtop_ref[...]
    lane_iota = lax.broadcasted_iota(jnp.int32, (SB, V), 1)

    hs = [jnp.zeros((SB, H), jnp.float32) for _ in range(_NCH)]
    cs = [jnp.zeros((SB, H), jnp.float32) for _ in range(_NCH)]

    for t in range(T):
        for ch in range(_NCH):
            rows = slice(ch * SB, (ch + 1) * SB)
            onehot = (idx[rows, t:t + 1] == lane_iota).astype(jnp.float32)
            lhs = jnp.concatenate([onehot, hs[ch]], axis=1)
            act = jnp.tanh(jnp.dot(lhs, wtop,
                                   preferred_element_type=jnp.float32))
            t_i = act[:, 0 * H:1 * H]
            t_f = act[:, 1 * H:2 * H]
            g_g = act[:, 2 * H:3 * H]
            t_o = act[:, 3 * H:4 * H]
            c = 0.5 * ((t_f * cs[ch] + cs[ch]) + (t_i * g_g + g_g))
            th = jnp.tanh(c)
            cs[ch] = c
            hs[ch] = 0.5 * (t_o * th + th)

    wout = wout_ref[...]
    blin = blin_ref[...]
    for ch in range(_NCH):
        rows = slice(ch * SB, (ch + 1) * SB)
        comb = jnp.concatenate([wemb_ref[rows, :], hs[ch]], axis=1)
        res = jnp.dot(comb, wout, preferred_element_type=jnp.float32) + blin
        out_ref[rows, :] = jnp.maximum(res, 0.0)


def kernel(char_indices, word_embedding, emb_tbl, w_ih, w_hh, bias,
           w_lw, w_lh, b_lin):
    B, T = char_indices.shape
    Dw = word_embedding.shape[1]
    H4 = w_ih.shape[0]
    H = H4 // 4
    V = emb_tbl.shape[0]

    colw = jnp.arange(H4)
    sc = jnp.where((colw >= 2 * H) & (colw < 3 * H), 1.0, 0.5)
    sc = sc.astype(jnp.float32)[None, :]

    folded = emb_tbl @ w_ih.T + bias
    wtop = jnp.concatenate([folded, w_hh.T], axis=0) * sc
    wout = jnp.concatenate([w_lw, w_lh], axis=1).T

    n_blk = (B + _BLOCK - 1) // _BLOCK
    Bp = n_blk * _BLOCK
    if Bp != B:
        char_indices = jnp.pad(char_indices, ((0, Bp - B), (0, 0)))
        word_embedding = jnp.pad(word_embedding, ((0, Bp - B), (0, 0)))

    out = pl.pallas_call(
        _encoder_kernel,
        out_shape=jax.ShapeDtypeStruct((Bp, Dw), jnp.float32),
        grid=(n_blk,),
        in_specs=[
            pl.BlockSpec((_BLOCK, T), lambda i: (i, 0)),
            pl.BlockSpec((_BLOCK, Dw), lambda i: (i, 0)),
            pl.BlockSpec((V + H, H4), lambda i: (0, 0)),
            pl.BlockSpec((Dw + H, Dw), lambda i: (0, 0)),
            pl.BlockSpec((1, Dw), lambda i: (0, 0)),
        ],
        out_specs=pl.BlockSpec((_BLOCK, Dw), lambda i: (i, 0)),
        compiler_params=pltpu.CompilerParams(
            dimension_semantics=("parallel",)),
    )(char_indices, word_embedding, wtop, wout, b_lin)
    return out if Bp == B else out[:B]


# speedup vs baseline: 4.6111x; 1.0393x over previous
import jax
import jax.numpy as jnp
from jax import lax
from jax.experimental import pallas as pl
from jax.experimental.pallas import tpu as pltpu

_BLOCK = 1024
_NCH = 4


def _encoder_kernel(idx_ref, wemb_ref, wtop_ref, wout_ref,
                    blin_ref, out_ref):
    BLK, T = idx_ref.shape
    VH, H4 = wtop_ref.shape
    H = H4 // 4
    V = VH - H
    Dw = out_ref.shape[1]
    SB = BLK // _NCH

    idx = idx_ref[...]
    wtop = wtop_ref[...]
    lane_iota = lax.broadcasted_iota(jnp.int32, (SB, V), 1)

    hs = [jnp.zeros((SB, H), jnp.float32) for _ in range(_NCH)]
    cs = [jnp.zeros((SB, H), jnp.float32) for _ in range(_NCH)]

    for t in range(T):
        for ch in range(_NCH):
            rows = slice(ch * SB, (ch + 1) * SB)
            onehot = (idx[rows, t:t + 1] == lane_iota).astype(jnp.float32)
            lhs = jnp.concatenate([onehot, hs[ch]], axis=1)
            act = jnp.tanh(jnp.dot(lhs, wtop,
                                   preferred_element_type=jnp.float32))
            t_i = act[:, 0 * H:1 * H]
            t_f = act[:, 1 * H:2 * H]
            g_g = act[:, 2 * H:3 * H]
            t_o = act[:, 3 * H:4 * H]
            c = 0.5 * ((t_f * cs[ch] + cs[ch]) + (t_i * g_g + g_g))
            th = jnp.tanh(c)
            cs[ch] = c
            hs[ch] = 0.5 * (t_o * th + th)

    wout = wout_ref[...]
    blin = blin_ref[...]
    for ch in range(_NCH):
        rows = slice(ch * SB, (ch + 1) * SB)
        comb = jnp.concatenate([wemb_ref[rows, :], hs[ch]], axis=1)
        res = jnp.dot(comb, wout, preferred_element_type=jnp.float32) + blin
        out_ref[rows, :] = jnp.maximum(res, 0.0)


def kernel(char_indices, word_embedding, emb_tbl, w_ih, w_hh, bias,
           w_lw, w_lh, b_lin):
    B, T = char_indices.shape
    Dw = word_embedding.shape[1]
    H4 = w_ih.shape[0]
    H = H4 // 4
    V = emb_tbl.shape[0]

    colw = jnp.arange(H4)
    sc = jnp.where((colw >= 2 * H) & (colw < 3 * H), 1.0, 0.5)
    sc = sc.astype(jnp.float32)[None, :]

    folded = emb_tbl @ w_ih.T + bias
    wtop = jnp.concatenate([folded, w_hh.T], axis=0) * sc
    wout = jnp.concatenate([w_lw, w_lh], axis=1).T

    n_blk = (B + _BLOCK - 1) // _BLOCK
    Bp = n_blk * _BLOCK
    if Bp != B:
        char_indices = jnp.pad(char_indices, ((0, Bp - B), (0, 0)))
        word_embedding = jnp.pad(word_embedding, ((0, Bp - B), (0, 0)))

    out = pl.pallas_call(
        _encoder_kernel,
        out_shape=jax.ShapeDtypeStruct((Bp, Dw), jnp.float32),
        grid=(n_blk,),
        in_specs=[
            pl.BlockSpec((_BLOCK, T), lambda i: (i, 0)),
            pl.BlockSpec((_BLOCK, Dw), lambda i: (i, 0)),
            pl.BlockSpec((V + H, H4), lambda i: (0, 0)),
            pl.BlockSpec((Dw + H, Dw), lambda i: (0, 0)),
            pl.BlockSpec((1, Dw), lambda i: (0, 0)),
        ],
        out_specs=pl.BlockSpec((_BLOCK, Dw), lambda i: (i, 0)),
        compiler_params=pltpu.CompilerParams(
            dimension_semantics=("parallel",)),
    )(char_indices, word_embedding, wtop, wout, b_lin)
    return out if Bp == B else out[:B]
